# baseline (speedup 1.0000x reference)
import jax
import jax.numpy as jnp
from jax import lax
from jax.experimental import pallas as pl
from jax.experimental.pallas import tpu as pltpu
from jax.experimental.pallas import tpu_sc as plsc

B = 4
N = 10000
P = 128
LRF = 16
NC = 2
NS = 16
NW = NC * NS
CPW = P // NW
CH = 400
NCHUNK = N // CH
GROUPS = CH // 16


def _sc_kernel(xt, idx2, kbx, yt, tbl0, tbl1, tbl2, tbl3, idx_a, idx_b,
               out_v, kbt_v, sem_t, sem_a, sem_b, sem_o):
    tbls = (tbl0, tbl1, tbl2, tbl3)
    wid = lax.axis_index("s") * NC + lax.axis_index("c")
    c0 = wid * CPW

    iota = lax.iota(jnp.int32, 16)
    colsk = [(iota + j) & (LRF - 1) for j in range(LRF)]

    def start_idx(c, t, buf, sem):
        pltpu.async_copy(
            idx2.at[pl.ds(t * CH, CH), pl.ds(c * LRF, LRF)], buf, sem)

    def wait_idx(buf, sem):
        pltpu.make_async_copy(
            idx2.at[pl.ds(0, CH), pl.ds(0, LRF)], buf, sem).wait()

    def do_chunk(t, buf):
        i0 = t * CH

        kjs = [kbt_v[j] for j in range(LRF)]
        bias_bc = kbt_v[LRF]

        def group_body(g, _):
            nb = g * 16
            rows = nb + iota
            accs = [bias_bc, bias_bc, bias_bc, bias_bc]
            for j in range(LRF):
                icol = plsc.load_gather(buf, [rows, colsk[j]])
                kj = kjs[j]
                for b in range(B):
                    vb = plsc.load_gather(tbls[b], [icol])
                    accs[b] = accs[b] + kj * vb
            for b in range(B):
                out_v[pl.ds(b * N + i0 + nb, 16)] = accs[b]
            return 0

        lax.fori_loop(0, GROUPS, group_body, 0)

    out_cp = None
    for ci in range(CPW):
        c = c0 + ci
        copies = [pltpu.async_copy(xt.at[pl.ds((c * B + b) * N, N)], tbls[b],
                                   sem_t)
                  for b in range(B)]
        copies.append(pltpu.async_copy(kbx.at[c], kbt_v, sem_t))
        start_idx(c, 0, idx_a, sem_a)
        for cp in copies:
            cp.wait()
        if out_cp is not None:
            out_cp.wait()

        def pair_body(p, _, c=c):
            start_idx(c, 2 * p + 1, idx_b, sem_b)
            wait_idx(idx_a, sem_a)
            do_chunk(2 * p, idx_a)
            start_idx(c, 2 * p + 2, idx_a, sem_a)
            wait_idx(idx_b, sem_b)
            do_chunk(2 * p + 1, idx_b)
            return 0

        lax.fori_loop(0, (NCHUNK - 1) // 2, pair_body, 0)
        wait_idx(idx_a, sem_a)
        do_chunk(NCHUNK - 1, idx_a)
        out_cp = pltpu.async_copy(out_v, yt.at[pl.ds(c * B * N, B * N)], sem_o)
    out_cp.wait()


@jax.jit
def kernel(x, lrf_idx, kernel, bias):
    xt = jnp.transpose(x, (2, 0, 1)).reshape(P * B * N)
    kt = jnp.transpose(kernel, (1, 0))
    jsk = (jnp.arange(LRF)[:, None] + jnp.arange(16)[None, :]) % LRF
    kbs = kt[:, jsk]
    kbx = jnp.concatenate(
        [kbs, jnp.broadcast_to(bias[:, None, None], (P, 1, 16))], axis=1)
    idx2 = lrf_idx.reshape(N, P * LRF)

    mesh = plsc.VectorSubcoreMesh(core_axis_name="c", subcore_axis_name="s")
    yt = pl.kernel(
        _sc_kernel,
        out_type=jax.ShapeDtypeStruct((P * B * N,), jnp.float32),
        mesh=mesh,
        scratch_types=[
            pltpu.VMEM((N,), jnp.float32),
            pltpu.VMEM((N,), jnp.float32),
            pltpu.VMEM((N,), jnp.float32),
            pltpu.VMEM((N,), jnp.float32),
            pltpu.VMEM((CH, LRF), jnp.int32),
            pltpu.VMEM((CH, LRF), jnp.int32),
            pltpu.VMEM((B * N,), jnp.float32),
            pltpu.VMEM((LRF + 1, 16), jnp.float32),
            pltpu.SemaphoreType.DMA,
            pltpu.SemaphoreType.DMA,
            pltpu.SemaphoreType.DMA,
            pltpu.SemaphoreType.DMA,
        ],
        compiler_params=pltpu.CompilerParams(
            use_tc_tiling_on_sc=False, needs_layout_passes=False),
    )(xt, idx2, kbx)
    return jnp.transpose(yt.reshape(P, B, N), (1, 2, 0))


# speedup vs baseline: 1.0401x; 1.0401x over previous
import jax
import jax.numpy as jnp
from jax import lax
from jax.experimental import pallas as pl
from jax.experimental.pallas import tpu as pltpu
from jax.experimental.pallas import tpu_sc as plsc

B = 4
N = 10000
P = 128
LRF = 16
NC = 2
NS = 16
NW = NC * NS
CPW = P // NW
CH = 400
NCHUNK = N // CH
GROUPS = CH // 16


def _sc_kernel(xt, idx2, kbx, yt, tbl0, tbl1, tbl2, tbl3, idx_a, idx_b,
               out_v, kbt_v, sem_t, sem_a, sem_b, sem_o):
    tbls = (tbl0, tbl1, tbl2, tbl3)
    wid = lax.axis_index("s") * NC + lax.axis_index("c")
    c0 = wid * CPW

    iota = lax.iota(jnp.int32, 16)
    colsk = [(iota + j) & (LRF - 1) for j in range(LRF)]

    def start_idx(c, t, buf, sem):
        pltpu.async_copy(
            idx2.at[pl.ds(t * CH, CH), pl.ds(c * LRF, LRF)], buf, sem)

    def wait_idx(buf, sem):
        pltpu.make_async_copy(
            idx2.at[pl.ds(0, CH), pl.ds(0, LRF)], buf, sem).wait()

    def do_chunk(t, buf):
        i0 = t * CH

        kjs = [kbt_v[j] for j in range(LRF)]
        bias_bc = kbt_v[LRF]

        def group_body(g, _):
            nb = g * 16
            rows = nb + iota
            zero = bias_bc * 0.0
            accs = [bias_bc, bias_bc, bias_bc, bias_bc]
            acc2 = [zero, zero, zero, zero]
            for j in range(LRF):
                icol = plsc.load_gather(buf, [rows, colsk[j]])
                kj = kjs[j]
                dst = accs if (j % 2 == 0) else acc2
                for b in range(B):
                    vb = plsc.load_gather(tbls[b], [icol])
                    dst[b] = dst[b] + kj * vb
            for b in range(B):
                out_v[pl.ds(b * N + i0 + nb, 16)] = accs[b] + acc2[b]
            return 0

        lax.fori_loop(0, GROUPS, group_body, 0)

    out_cp = None
    for ci in range(CPW):
        c = c0 + ci
        copies = [pltpu.async_copy(xt.at[c, b], tbls[b], sem_t)
                  for b in range(B)]
        copies.append(pltpu.async_copy(kbx.at[c], kbt_v, sem_t))
        start_idx(c, 0, idx_a, sem_a)
        for cp in copies:
            cp.wait()
        if out_cp is not None:
            out_cp.wait()

        def pair_body(p, _, c=c):
            start_idx(c, 2 * p + 1, idx_b, sem_b)
            wait_idx(idx_a, sem_a)
            do_chunk(2 * p, idx_a)
            start_idx(c, 2 * p + 2, idx_a, sem_a)
            wait_idx(idx_b, sem_b)
            do_chunk(2 * p + 1, idx_b)
            return 0

        lax.fori_loop(0, (NCHUNK - 1) // 2, pair_body, 0)
        wait_idx(idx_a, sem_a)
        do_chunk(NCHUNK - 1, idx_a)
        out_cp = pltpu.async_copy(out_v, yt.at[c], sem_o)
    out_cp.wait()


@jax.jit
def kernel(x, lrf_idx, kernel, bias):
    xt = jnp.transpose(x, (2, 0, 1))
    kt = jnp.transpose(kernel, (1, 0))
    jsk = (jnp.arange(LRF)[:, None] + jnp.arange(16)[None, :]) % LRF
    kbs = kt[:, jsk]
    kbx = jnp.concatenate(
        [kbs, jnp.broadcast_to(bias[:, None, None], (P, 1, 16))], axis=1)
    idx2 = lrf_idx.reshape(N, P * LRF)

    mesh = plsc.VectorSubcoreMesh(core_axis_name="c", subcore_axis_name="s")
    yt = pl.kernel(
        _sc_kernel,
        out_type=jax.ShapeDtypeStruct((P, B * N), jnp.float32),
        mesh=mesh,
        scratch_types=[
            pltpu.VMEM((N,), jnp.float32),
            pltpu.VMEM((N,), jnp.float32),
            pltpu.VMEM((N,), jnp.float32),
            pltpu.VMEM((N,), jnp.float32),
            pltpu.VMEM((CH, LRF), jnp.int32),
            pltpu.VMEM((CH, LRF), jnp.int32),
            pltpu.VMEM((B * N,), jnp.float32),
            pltpu.VMEM((LRF + 1, 16), jnp.float32),
            pltpu.SemaphoreType.DMA,
            pltpu.SemaphoreType.DMA,
            pltpu.SemaphoreType.DMA,
            pltpu.SemaphoreType.DMA,
        ],
        compiler_params=pltpu.CompilerParams(
            use_tc_tiling_on_sc=False, needs_layout_passes=False),
    )(xt, idx2, kbx)
    return jnp.transpose(yt.reshape(P, B, N), (1, 2, 0))


# speedup vs baseline: 1.0528x; 1.0122x over previous
import jax
import jax.numpy as jnp
from jax import lax
from jax.experimental import pallas as pl
from jax.experimental.pallas import tpu as pltpu
from jax.experimental.pallas import tpu_sc as plsc

B = 4
N = 10000
P = 128
LRF = 16
NC = 2
NS = 16
NW = NC * NS
CPW = P // NW
CH = 400
NCHUNK = N // CH
GROUPS = CH // 16


def _sc_kernel(xt, idx2, kbx, yt, tbl0, tbl1, tbl2, tbl3, idx_a, idx_b,
               out_v, kbt_v, sem_t, sem_a, sem_b, sem_o):
    tbls = (tbl0, tbl1, tbl2, tbl3)
    wid = lax.axis_index("s") * NC + lax.axis_index("c")
    c0 = wid * CPW

    iota = lax.iota(jnp.int32, 16)
    colsk = [(iota + j) & (LRF - 1) for j in range(LRF)]

    def start_idx(c, t, buf, sem):
        pltpu.async_copy(
            idx2.at[pl.ds(t * CH, CH), pl.ds(c * LRF, LRF)], buf, sem)

    def wait_idx(buf, sem):
        pltpu.make_async_copy(
            idx2.at[pl.ds(0, CH), pl.ds(0, LRF)], buf, sem).wait()

    def do_chunk(t, buf):
        i0 = t * CH

        kjs = [kbt_v[j] for j in range(LRF)]
        bias_bc = kbt_v[LRF]

        def group_body(g, _):
            nb = g * 16
            rows = nb + iota
            accs = [bias_bc, bias_bc, bias_bc, bias_bc]
            for j in range(LRF):
                icol = plsc.load_gather(buf, [rows, colsk[j]])
                kj = kjs[j]
                for b in range(B):
                    vb = plsc.load_gather(tbls[b], [icol])
                    accs[b] = accs[b] + kj * vb
            for b in range(B):
                out_v[pl.ds(b * N + i0 + nb, 16)] = accs[b]
            return 0

        lax.fori_loop(0, GROUPS, group_body, 0)

    out_cp = None
    for ci in range(CPW):
        c = c0 + ci
        copies = [pltpu.async_copy(xt.at[c, b], tbls[b], sem_t)
                  for b in range(B)]
        copies.append(pltpu.async_copy(kbx.at[c], kbt_v, sem_t))
        start_idx(c, 0, idx_a, sem_a)
        for cp in copies:
            cp.wait()
        if out_cp is not None:
            out_cp.wait()

        def pair_body(p, _, c=c):
            start_idx(c, 2 * p + 1, idx_b, sem_b)
            wait_idx(idx_a, sem_a)
            do_chunk(2 * p, idx_a)
            start_idx(c, 2 * p + 2, idx_a, sem_a)
            wait_idx(idx_b, sem_b)
            do_chunk(2 * p + 1, idx_b)
            return 0

        lax.fori_loop(0, (NCHUNK - 1) // 2, pair_body, 0)
        wait_idx(idx_a, sem_a)
        do_chunk(NCHUNK - 1, idx_a)
        out_cp = pltpu.async_copy(out_v, yt.at[c], sem_o)
    out_cp.wait()


@jax.jit
def kernel(x, lrf_idx, kernel, bias):
    xt = jnp.transpose(x, (2, 0, 1))
    kt = jnp.transpose(kernel, (1, 0))
    jsk = (jnp.arange(LRF)[:, None] + jnp.arange(16)[None, :]) % LRF
    kbs = kt[:, jsk]
    kbx = jnp.concatenate(
        [kbs, jnp.broadcast_to(bias[:, None, None], (P, 1, 16))], axis=1)
    idx2 = lrf_idx.reshape(N, P * LRF)

    mesh = plsc.VectorSubcoreMesh(core_axis_name="c", subcore_axis_name="s")
    yt = pl.kernel(
        _sc_kernel,
        out_type=jax.ShapeDtypeStruct((P, B * N), jnp.float32),
        mesh=mesh,
        scratch_types=[
            pltpu.VMEM((N,), jnp.float32),
            pltpu.VMEM((N,), jnp.float32),
            pltpu.VMEM((N,), jnp.float32),
            pltpu.VMEM((N,), jnp.float32),
            pltpu.VMEM((CH, LRF), jnp.int32),
            pltpu.VMEM((CH, LRF), jnp.int32),
            pltpu.VMEM((B * N,), jnp.float32),
            pltpu.VMEM((LRF + 1, 16), jnp.float32),
            pltpu.SemaphoreType.DMA,
            pltpu.SemaphoreType.DMA,
            pltpu.SemaphoreType.DMA,
            pltpu.SemaphoreType.DMA,
        ],
        compiler_params=pltpu.CompilerParams(
            use_tc_tiling_on_sc=False, needs_layout_passes=False),
    )(xt, idx2, kbx)
    return jnp.transpose(yt.reshape(P, B, N), (1, 2, 0))
